# Initial kernel scaffold; baseline (speedup 1.0000x reference)
import jax
import jax.numpy as jnp
from jax.experimental import pallas as pl


def kernel(pred, target):
    raise NotImplementedError("write your pallas kernel here")


# speedup vs baseline: 2.4136x; 2.4136x over previous
import functools

import jax
import jax.numpy as jnp
from jax import lax
from jax.experimental import pallas as pl
from jax.experimental.pallas import tpu as pltpu
from jax.experimental.pallas import tpu_sc as plsc

_ALPHA = 0.1
_N = 16384
_C = 1000
_NT = 16
_CHUNK = _N // _NT
_BINS = 1024
_PER = _BINS // _NT


@functools.partial(
    pl.kernel,
    out_type=[
        jax.ShapeDtypeStruct((_BINS,), jnp.int32),
        jax.ShapeDtypeStruct((_BINS, _C), jnp.float32),
    ],
    mesh=plsc.VectorSubcoreMesh(
        core_axis_name="c", subcore_axis_name="s", num_cores=1
    ),
    scratch_types=[
        pltpu.VMEM((_CHUNK,), jnp.int32),
        pltpu.VMEM((_BINS,), jnp.int32),
        pltpu.VMEM_SHARED((_NT, _BINS), jnp.int32),
        pltpu.VMEM((_NT, _BINS), jnp.int32),
        pltpu.VMEM((_PER,), jnp.int32),
        pltpu.VMEM((_PER,), jnp.int32),
        pltpu.VMEM((_PER, _C), jnp.float32),
        pltpu.SemaphoreType.DMA,
    ],
    compiler_params=pltpu.CompilerParams(
        needs_layout_passes=False, use_tc_tiling_on_sc=False
    ),
)
def _sc_winner_gather(target_hbm, pred_hbm, win_hbm, gth_hbm,
                      tgt_v, bins_v, shared, allbins_v, win_v, idx_v, rows_v,
                      sem):
    s = lax.axis_index("s")
    base = s * _CHUNK
    pltpu.sync_copy(target_hbm.at[pl.ds(base, _CHUNK)], tgt_v)

    neg1 = jnp.full((16,), -1, jnp.int32)
    for k in range(_BINS // 16):
        bins_v[pl.ds(k * 16, 16)] = neg1

    lanes = lax.iota(jnp.int32, 16)

    def body(k, carry):
        tv = tgt_v[pl.ds(k * 16, 16)]
        vals = (base + k * 16) + lanes
        for j in range(16):
            plsc.store_scatter(bins_v, [tv], vals, mask=lanes == j)
        return carry

    lax.fori_loop(0, _CHUNK // 16, body, 0)

    pltpu.sync_copy(bins_v, shared.at[s])
    plsc.subcore_barrier()
    pltpu.sync_copy(shared, allbins_v)

    cbase = s * _PER
    for c in range(_PER // 16):
        off = cbase + c * 16
        acc = allbins_v[0, pl.ds(off, 16)]
        for r in range(1, _NT):
            acc = jnp.maximum(acc, allbins_v[r, pl.ds(off, 16)])
        win_v[pl.ds(c * 16, 16)] = acc
        idx_v[pl.ds(c * 16, 16)] = jnp.maximum(acc, 0)

    pltpu.sync_copy(win_v, win_hbm.at[pl.ds(cbase, _PER)])
    pltpu.async_copy(pred_hbm.at[idx_v], rows_v, sem).wait()
    pltpu.sync_copy(rows_v, gth_hbm.at[pl.ds(cbase, _PER)])


_BR = 256


def _dense_body(x_ref, f_ref, lse_ref):
    x = x_ref[...]
    m = jnp.max(x, axis=1, keepdims=True)
    e = jnp.exp(x - m)
    srow = jnp.sum(e, axis=1, keepdims=True)
    p = e / srow
    f_ref[...] = jnp.sum(-_ALPHA * p * p * jnp.log(1.0 - p), axis=1)
    lse_ref[...] = m[:, 0] + jnp.log(srow[:, 0])


_dense = pl.pallas_call(
    _dense_body,
    grid=(_N // _BR,),
    in_specs=[pl.BlockSpec((_BR, _C), lambda i: (i, 0))],
    out_specs=[
        pl.BlockSpec((_BR,), lambda i: (i,)),
        pl.BlockSpec((_BR,), lambda i: (i,)),
    ],
    out_shape=[
        jax.ShapeDtypeStruct((_N,), jnp.float32),
        jax.ShapeDtypeStruct((_N,), jnp.float32),
    ],
)


def _corr_body(x_ref, g_ref, w_ref, f_ref, lse_ref, out_ref):
    x = x_ref[...]
    g = g_ref[...]
    w = w_ref[...]
    f = f_ref[...]
    lse = lse_ref[...]

    lse_h = lse[:_BINS]
    p2 = jnp.exp(2.0 * (x - lse_h[:, None]))
    mg = jnp.max(g, axis=1)
    sg = jnp.sum(jnp.exp(g - mg[:, None]), axis=1)
    lse_g = mg + jnp.log(sg)
    gdot = jnp.sum(p2 * g, axis=1)
    s2 = jnp.sum(p2, axis=1)
    gterm = -(1.0 - _ALPHA) * (gdot - lse_g * s2)

    hit = w >= 0
    head = jnp.where(hit, gterm, f[:_BINS])
    out_ref[0, 0] = jnp.sum(head) + jnp.sum(f[_BINS:])


_corr = pl.pallas_call(
    _corr_body,
    grid=(1,),
    in_specs=[
        pl.BlockSpec((_BINS, _C), lambda i: (0, 0)),
        pl.BlockSpec((_BINS, _C), lambda i: (0, 0)),
        pl.BlockSpec((_BINS,), lambda i: (0,)),
        pl.BlockSpec((_N,), lambda i: (0,)),
        pl.BlockSpec((_N,), lambda i: (0,)),
    ],
    out_specs=pl.BlockSpec((1, 1), lambda i: (0, 0), memory_space=pltpu.SMEM),
    out_shape=jax.ShapeDtypeStruct((1, 1), jnp.float32),
)


def kernel(pred, target):
    target = target.astype(jnp.int32)
    win, gth = _sc_winner_gather(target, pred)
    f, lse = _dense(pred)
    total = _corr(pred, gth, win, f, lse)
    return total[0, 0]
